# baseline (speedup 1.0000x reference)
import functools

import jax
import jax.numpy as jnp
from jax import lax
from jax.experimental import pallas as pl
from jax.experimental.pallas import tpu as pltpu
from jax.experimental.pallas import tpu_sc as plsc

N = 10000
D = 128
G = 8
E = 320000

NC, NS, L = 2, 16, 16
NW = NC * NS
CHUNK = 128
NCH = 79
EPW = NCH * CHUNK
EPAD = NW * EPW
NCHT = EPAD // CHUNK
NCH0 = 55
NCH1 = 2 * NCH - NCH0
NPAD = 10240
RPS = NPAD // NS

BM = 2000


def _deg_body(eidx_hbm, out_hbm, ib, acc_t):
    c = lax.axis_index("c")
    s = lax.axis_index("s")
    gw = c * NS + s

    pltpu.sync_copy(eidx_hbm.at[pl.ds(gw * NCH, NCH)], ib)

    one = jnp.ones((L,), jnp.float32)
    zero = jnp.zeros((L,), jnp.float32)

    def zero_acc(i, carry):
        acc_t[i, :] = zero
        return carry

    lax.fori_loop(0, NPAD // L, zero_acc, 0)

    def edge_body(g, carry):
        for k in range(CHUNK // L):
            idx = ib[g, 1, pl.ds(k * L, L)]
            row = lax.shift_right_logical(idx, 4)
            col = lax.bitwise_and(idx, 15)
            plsc.addupdate_scatter(acc_t, [row, col], one)
        return carry

    lax.fori_loop(0, NCH, edge_body, 0)

    pltpu.sync_copy(acc_t, out_hbm.at[gw])


def _segsum_body(eidx_hbm, h_hbm, out_hbm,
                 ib0, ib1, ib2, rows0, rows1, z_v, acc_sh,
                 sem_g0, sem_g1, sem_s0, sem_s1, sem_i0, sem_i1, sem_i2):
    c = lax.axis_index("c")
    s = lax.axis_index("s")
    gw = c * NS + s

    ib = (ib0, ib1, ib2)
    rows = (rows0, rows1)
    sem_g = (sem_g0, sem_g1)
    sem_s = (sem_s0, sem_s1)
    sem_i = (sem_i0, sem_i1, sem_i2)

    def gather_chunk(b, i3, sem_b):
        pltpu.async_copy(h_hbm.at[ib[i3].at[0]], rows[b], sem_b)

    def drain_gather(b, i3, sem_b):
        pltpu.make_async_copy(h_hbm.at[ib[i3].at[0]], rows[b], sem_b).wait()

    def scatter_chunk(b, i3, sem_b):
        pltpu.async_copy(rows[b], acc_sh.at[ib[i3].at[1]], sem_b, add=True)

    def drain_scatter(b, i3, sem_b):
        pltpu.make_async_copy(rows[b], acc_sh.at[ib[i3].at[1]], sem_b).wait()

    def fetch_idx(g, i3, sem_b):
        pltpu.async_copy(eidx_hbm.at[g], ib[i3], sem_b)

    def drain_idx(g, i3, sem_b):
        pltpu.make_async_copy(eidx_hbm.at[g], ib[i3], sem_b).wait()

    def run_pipeline(base, count):
        pltpu.sync_copy(eidx_hbm.at[base], ib[0])
        gather_chunk(0, 0, sem_g[0])
        fetch_idx(base + 1, 1, sem_i[1])

        zero = jnp.zeros((L,), jnp.float32)
        for i in range(16):
            for j in range(D // L):
                z_v[i, pl.ds(j * L, L)] = zero

        def zero_body(j, carry):
            pltpu.sync_copy(z_v, acc_sh.at[pl.ds(s * RPS + j * 16, 16)])
            return carry

        lax.fori_loop(0, RPS // 16, zero_body, 0)
        plsc.subcore_barrier()

        def half(t, b, i3):
            drain_gather(b, i3, sem_g[b])

            @pl.when(t > 0)
            def _():
                drain_scatter(1 - b, (i3 + 2) % 3, sem_s[1 - b])

            @pl.when(t + 1 < count)
            def _():
                drain_idx(base + t + 1, (i3 + 1) % 3, sem_i[(i3 + 1) % 3])
                gather_chunk(1 - b, (i3 + 1) % 3, sem_g[1 - b])

            scatter_chunk(b, i3, sem_s[b])

            @pl.when(t + 2 < count)
            def _():
                fetch_idx(base + t + 2, (i3 + 2) % 3, sem_i[(i3 + 2) % 3])

        def six(p, carry):
            for u in range(6):
                half(6 * p + u, u % 2, u % 3)
            return carry

        lax.fori_loop(0, count // 6, six, 0)
        half(count - 1, 0, 0)
        drain_scatter(0, 0, sem_s[0])
        plsc.subcore_barrier()

    @pl.when(c == 0)
    def _():
        run_pipeline(s * NCH0, NCH0)

    @pl.when(c == 1)
    def _():
        run_pipeline(NS * NCH0 + s * NCH1, NCH1)

    pltpu.sync_copy(acc_sh.at[pl.ds(s * RPS, RPS)],
                    out_hbm.at[c, pl.ds(s * RPS, RPS)])


@functools.cache
def _sc_kernels():
    mesh = plsc.VectorSubcoreMesh(core_axis_name="c", subcore_axis_name="s",
                                  num_cores=NC, num_subcores=NS)
    deg = pl.kernel(
        _deg_body,
        out_type=jax.ShapeDtypeStruct((NW, NPAD // L, L), jnp.float32),
        mesh=mesh,
        compiler_params=pltpu.CompilerParams(needs_layout_passes=False),
        scratch_types=[
            pltpu.VMEM((NCH, 2, CHUNK), jnp.int32),
            pltpu.VMEM((NPAD // L, L), jnp.float32),
        ],
    )
    seg = pl.kernel(
        _segsum_body,
        out_type=jax.ShapeDtypeStruct((NC, NPAD, D), jnp.float32),
        mesh=mesh,
        scratch_types=[
            pltpu.VMEM((2, CHUNK), jnp.int32),
            pltpu.VMEM((2, CHUNK), jnp.int32),
            pltpu.VMEM((2, CHUNK), jnp.int32),
            pltpu.VMEM((CHUNK, D), jnp.float32),
            pltpu.VMEM((CHUNK, D), jnp.float32),
            pltpu.VMEM((16, D), jnp.float32),
            pltpu.VMEM_SHARED((NPAD, D), jnp.float32),
            pltpu.SemaphoreType.DMA,
            pltpu.SemaphoreType.DMA,
            pltpu.SemaphoreType.DMA,
            pltpu.SemaphoreType.DMA,
            pltpu.SemaphoreType.DMA,
            pltpu.SemaphoreType.DMA,
            pltpu.SemaphoreType.DMA,
        ],
    )
    return deg, seg


def _degsum_body(dp_ref, o_ref):
    o_ref[...] = jnp.sum(dp_ref[...], axis=0)


def _degsum(degp):
    return pl.pallas_call(
        _degsum_body,
        out_shape=jax.ShapeDtypeStruct((NPAD,), jnp.float32),
    )(degp)


def _dis_from(dp_ref):
    deg = dp_ref[...] + 1.0
    return lax.rsqrt(deg)


def _mm_scale_body(x_ref, dp_ref, w_ref, o_ref):
    dis = _dis_from(dp_ref)
    o_ref[...] = jnp.dot(x_ref[...], w_ref[...],
                         preferred_element_type=jnp.float32) * dis


def _layer_body(ap_ref, hp_ref, dp_ref, b_ref, w_ref, o_ref):
    dis = _dis_from(dp_ref)
    t = jnp.maximum((ap_ref[0] + ap_ref[1] + hp_ref[...]) * dis + b_ref[...],
                    0.0)
    o_ref[...] = jnp.dot(t, w_ref[...],
                         preferred_element_type=jnp.float32) * dis


def _heads_body(ap_ref, hp_ref, dp_ref, b_ref,
                wpi_ref, wmu_ref, wls_ref, bpi_ref, bmu_ref, bls_ref,
                opi_ref, omu_ref, ols_ref):
    dis = _dis_from(dp_ref)
    h = jnp.maximum((ap_ref[0] + ap_ref[1] + hp_ref[...]) * dis + b_ref[...],
                    0.0)
    zpi = jnp.dot(h, wpi_ref[...], preferred_element_type=jnp.float32) + bpi_ref[...]
    m = jnp.max(zpi, axis=1, keepdims=True)
    e = jnp.exp(zpi - m)
    opi_ref[...] = e / jnp.sum(e, axis=1, keepdims=True)
    omu_ref[...] = jnp.dot(h, wmu_ref[...], preferred_element_type=jnp.float32) + bmu_ref[...]
    ols_ref[...] = jnp.dot(h, wls_ref[...], preferred_element_type=jnp.float32) + bls_ref[...]


def _row_spec(width):
    return pl.BlockSpec((BM, width), lambda j: (j, 0))


_dp_spec = pl.BlockSpec((BM, 1), lambda j: (j, 0))
_ap_spec = pl.BlockSpec((NC, BM, D), lambda j: (0, j, 0))


def _full_spec(shape):
    return pl.BlockSpec(shape, lambda j: tuple(0 for _ in shape))


def _mm_scale(x, degp, w_t):
    return pl.pallas_call(
        _mm_scale_body,
        grid=(N // BM,),
        in_specs=[_row_spec(D), _dp_spec, _full_spec((D, D))],
        out_specs=_row_spec(D),
        out_shape=jax.ShapeDtypeStruct((N, D), jnp.float32),
    )(x, degp, w_t)


def _layer_finish(ap, hp, degp, b, w_t):
    return pl.pallas_call(
        _layer_body,
        grid=(N // BM,),
        in_specs=[_ap_spec, _row_spec(D), _dp_spec,
                  _full_spec((1, D)), _full_spec((D, D))],
        out_specs=_row_spec(D),
        out_shape=jax.ShapeDtypeStruct((N, D), jnp.float32),
    )(ap, hp, degp, b, w_t)


def _heads(ap, hp, degp, b, wpi_t, wmu_t, wls_t, bpi, bmu, bls):
    out = jax.ShapeDtypeStruct((N, G), jnp.float32)
    return pl.pallas_call(
        _heads_body,
        grid=(N // BM,),
        in_specs=[_ap_spec, _row_spec(D), _dp_spec, _full_spec((1, D)),
                  _full_spec((D, G)), _full_spec((D, G)), _full_spec((D, G)),
                  _full_spec((1, G)), _full_spec((1, G)), _full_spec((1, G))],
        out_specs=[_row_spec(G), _row_spec(G), _row_spec(G)],
        out_shape=[out, out, out],
    )(ap, hp, degp, b, wpi_t, wmu_t, wls_t, bpi, bmu, bls)


def kernel(x, edge_index, W1, b1, W2, b2, Wpi, bpi, Wmu, bmu, Wls, bls):
    ei = edge_index.astype(jnp.int32)
    pad = EPAD - E
    src = jnp.concatenate([ei[0], jnp.zeros((pad,), jnp.int32)])
    dst = jnp.concatenate([ei[1], jnp.full((pad,), N, jnp.int32)])

    eidx = jnp.stack([src.reshape(NCHT, CHUNK),
                      dst.reshape(NCHT, CHUNK)], axis=1)

    deg_kernel, segsum_kernel = _sc_kernels()
    degt = deg_kernel(eidx).reshape(NW, NPAD)
    degp = _degsum(degt)[:N, None]
    h1p = _mm_scale(x, degp, W1.T)
    a1 = segsum_kernel(eidx, h1p)
    h2p = _layer_finish(a1, h1p, degp, b1.reshape(1, D), W2.T)
    a2 = segsum_kernel(eidx, h2p)
    pi, mu, ls = _heads(a2, h2p, degp, b2.reshape(1, D),
                        Wpi.T, Wmu.T, Wls.T,
                        bpi.reshape(1, G), bmu.reshape(1, G), bls.reshape(1, G))
    return (pi, mu, ls)


# speedup vs baseline: 1.1700x; 1.1700x over previous
import functools

import jax
import jax.numpy as jnp
from jax import lax
from jax.experimental import pallas as pl
from jax.experimental.pallas import tpu as pltpu
from jax.experimental.pallas import tpu_sc as plsc

N = 10000
D = 128
G = 8
E = 320000

NC, NS, L = 2, 16, 16
NW = NC * NS
CHUNK = 128
NCH = 79
EPW = NCH * CHUNK
EPAD = NW * EPW
NCHT = EPAD // CHUNK
NCH0 = 103
NCH1 = 2 * NCH - NCH0
NPAD = 10240
RPS = NPAD // NS

BM = 2000


def _deg_body(eidx_hbm, out_hbm, ib, acc_t):
    c = lax.axis_index("c")
    s = lax.axis_index("s")
    gw = c * NS + s

    pltpu.sync_copy(eidx_hbm.at[pl.ds(gw * NCH, NCH)], ib)

    one = jnp.ones((L,), jnp.float32)
    zero = jnp.zeros((L,), jnp.float32)

    def zero_acc(i, carry):
        acc_t[i, :] = zero
        return carry

    lax.fori_loop(0, NPAD // L, zero_acc, 0)

    def edge_body(g, carry):
        for k in range(CHUNK // L):
            idx = ib[g, 1, pl.ds(k * L, L)]
            row = lax.shift_right_logical(idx, 4)
            col = lax.bitwise_and(idx, 15)
            plsc.addupdate_scatter(acc_t, [row, col], one)
        return carry

    lax.fori_loop(0, NCH, edge_body, 0)

    pltpu.sync_copy(acc_t, out_hbm.at[gw])


def _segsum_body(eidx_hbm, h_hbm, out_hbm,
                 ib0, ib1, ib2, rows0, rows1, z_v, acc_sh,
                 sem_g0, sem_g1, sem_s0, sem_s1, sem_i0, sem_i1, sem_i2):
    c = lax.axis_index("c")
    s = lax.axis_index("s")
    gw = c * NS + s

    ib = (ib0, ib1, ib2)
    rows = (rows0, rows1)
    sem_g = (sem_g0, sem_g1)
    sem_s = (sem_s0, sem_s1)
    sem_i = (sem_i0, sem_i1, sem_i2)

    def gather_chunk(b, i3, sem_b):
        pltpu.async_copy(h_hbm.at[ib[i3].at[0]], rows[b], sem_b)

    def drain_gather(b, i3, sem_b):
        pltpu.make_async_copy(h_hbm.at[ib[i3].at[0]], rows[b], sem_b).wait()

    def scatter_chunk(b, i3, sem_b):
        pltpu.async_copy(rows[b], acc_sh.at[ib[i3].at[1]], sem_b, add=True)

    def drain_scatter(b, i3, sem_b):
        pltpu.make_async_copy(rows[b], acc_sh.at[ib[i3].at[1]], sem_b).wait()

    def fetch_idx(g, i3, sem_b):
        pltpu.async_copy(eidx_hbm.at[g], ib[i3], sem_b)

    def drain_idx(g, i3, sem_b):
        pltpu.make_async_copy(eidx_hbm.at[g], ib[i3], sem_b).wait()

    def run_pipeline(base, count):
        pltpu.sync_copy(eidx_hbm.at[base], ib[0])
        gather_chunk(0, 0, sem_g[0])
        fetch_idx(base + 1, 1, sem_i[1])

        zero = jnp.zeros((L,), jnp.float32)
        for i in range(16):
            for j in range(D // L):
                z_v[i, pl.ds(j * L, L)] = zero

        def zero_body(j, carry):
            pltpu.sync_copy(z_v, acc_sh.at[pl.ds(s * RPS + j * 16, 16)])
            return carry

        lax.fori_loop(0, RPS // 16, zero_body, 0)
        plsc.subcore_barrier()

        def half(t, b, i3):
            drain_gather(b, i3, sem_g[b])

            @pl.when(t > 0)
            def _():
                drain_scatter(1 - b, (i3 + 2) % 3, sem_s[1 - b])

            @pl.when(t + 1 < count)
            def _():
                drain_idx(base + t + 1, (i3 + 1) % 3, sem_i[(i3 + 1) % 3])
                gather_chunk(1 - b, (i3 + 1) % 3, sem_g[1 - b])

            scatter_chunk(b, i3, sem_s[b])

            @pl.when(t + 2 < count)
            def _():
                fetch_idx(base + t + 2, (i3 + 2) % 3, sem_i[(i3 + 2) % 3])

        def six(p, carry):
            for u in range(6):
                half(6 * p + u, u % 2, u % 3)
            return carry

        lax.fori_loop(0, count // 6, six, 0)
        half(count - 1, 0, 0)
        drain_scatter(0, 0, sem_s[0])
        plsc.subcore_barrier()

    @pl.when(c == 0)
    def _():
        run_pipeline(s * NCH0, NCH0)

    @pl.when(c == 1)
    def _():
        run_pipeline(NS * NCH0 + s * NCH1, NCH1)

    pltpu.sync_copy(acc_sh.at[pl.ds(s * RPS, RPS)],
                    out_hbm.at[c, pl.ds(s * RPS, RPS)])


@functools.cache
def _sc_kernels():
    mesh = plsc.VectorSubcoreMesh(core_axis_name="c", subcore_axis_name="s",
                                  num_cores=NC, num_subcores=NS)
    deg = pl.kernel(
        _deg_body,
        out_type=jax.ShapeDtypeStruct((NW, NPAD // L, L), jnp.float32),
        mesh=mesh,
        compiler_params=pltpu.CompilerParams(needs_layout_passes=False),
        scratch_types=[
            pltpu.VMEM((NCH, 2, CHUNK), jnp.int32),
            pltpu.VMEM((NPAD // L, L), jnp.float32),
        ],
    )
    seg = pl.kernel(
        _segsum_body,
        out_type=jax.ShapeDtypeStruct((NC, NPAD, D), jnp.float32),
        mesh=mesh,
        scratch_types=[
            pltpu.VMEM((2, CHUNK), jnp.int32),
            pltpu.VMEM((2, CHUNK), jnp.int32),
            pltpu.VMEM((2, CHUNK), jnp.int32),
            pltpu.VMEM((CHUNK, D), jnp.float32),
            pltpu.VMEM((CHUNK, D), jnp.float32),
            pltpu.VMEM((16, D), jnp.float32),
            pltpu.VMEM_SHARED((NPAD, D), jnp.float32),
            pltpu.SemaphoreType.DMA,
            pltpu.SemaphoreType.DMA,
            pltpu.SemaphoreType.DMA,
            pltpu.SemaphoreType.DMA,
            pltpu.SemaphoreType.DMA,
            pltpu.SemaphoreType.DMA,
            pltpu.SemaphoreType.DMA,
        ],
    )
    return deg, seg


def _degsum_body(dp_ref, o_ref):
    o_ref[...] = jnp.sum(dp_ref[...], axis=0)


def _degsum(degp):
    return pl.pallas_call(
        _degsum_body,
        out_shape=jax.ShapeDtypeStruct((NPAD,), jnp.float32),
    )(degp)


def _dis_from(dp_ref):
    deg = dp_ref[...] + 1.0
    return lax.rsqrt(deg)


def _mm_scale_body(x_ref, dp_ref, w_ref, o_ref):
    dis = _dis_from(dp_ref)
    o_ref[...] = jnp.dot(x_ref[...], w_ref[...],
                         preferred_element_type=jnp.float32) * dis


def _layer_body(ap_ref, hp_ref, dp_ref, b_ref, w_ref, o_ref):
    dis = _dis_from(dp_ref)
    t = jnp.maximum((ap_ref[0] + ap_ref[1] + hp_ref[...]) * dis + b_ref[...],
                    0.0)
    o_ref[...] = jnp.dot(t, w_ref[...],
                         preferred_element_type=jnp.float32) * dis


def _heads_body(ap_ref, hp_ref, dp_ref, b_ref,
                wpi_ref, wmu_ref, wls_ref, bpi_ref, bmu_ref, bls_ref,
                opi_ref, omu_ref, ols_ref):
    dis = _dis_from(dp_ref)
    h = jnp.maximum((ap_ref[0] + ap_ref[1] + hp_ref[...]) * dis + b_ref[...],
                    0.0)
    zpi = jnp.dot(h, wpi_ref[...], preferred_element_type=jnp.float32) + bpi_ref[...]
    m = jnp.max(zpi, axis=1, keepdims=True)
    e = jnp.exp(zpi - m)
    opi_ref[...] = e / jnp.sum(e, axis=1, keepdims=True)
    omu_ref[...] = jnp.dot(h, wmu_ref[...], preferred_element_type=jnp.float32) + bmu_ref[...]
    ols_ref[...] = jnp.dot(h, wls_ref[...], preferred_element_type=jnp.float32) + bls_ref[...]


def _row_spec(width):
    return pl.BlockSpec((BM, width), lambda j: (j, 0))


_dp_spec = pl.BlockSpec((BM, 1), lambda j: (j, 0))
_ap_spec = pl.BlockSpec((NC, BM, D), lambda j: (0, j, 0))


def _full_spec(shape):
    return pl.BlockSpec(shape, lambda j: tuple(0 for _ in shape))


def _mm_scale(x, degp, w_t):
    return pl.pallas_call(
        _mm_scale_body,
        grid=(N // BM,),
        in_specs=[_row_spec(D), _dp_spec, _full_spec((D, D))],
        out_specs=_row_spec(D),
        out_shape=jax.ShapeDtypeStruct((N, D), jnp.float32),
    )(x, degp, w_t)


def _layer_finish(ap, hp, degp, b, w_t):
    return pl.pallas_call(
        _layer_body,
        grid=(N // BM,),
        in_specs=[_ap_spec, _row_spec(D), _dp_spec,
                  _full_spec((1, D)), _full_spec((D, D))],
        out_specs=_row_spec(D),
        out_shape=jax.ShapeDtypeStruct((N, D), jnp.float32),
    )(ap, hp, degp, b, w_t)


def _heads(ap, hp, degp, b, wpi_t, wmu_t, wls_t, bpi, bmu, bls):
    out = jax.ShapeDtypeStruct((N, G), jnp.float32)
    return pl.pallas_call(
        _heads_body,
        grid=(N // BM,),
        in_specs=[_ap_spec, _row_spec(D), _dp_spec, _full_spec((1, D)),
                  _full_spec((D, G)), _full_spec((D, G)), _full_spec((D, G)),
                  _full_spec((1, G)), _full_spec((1, G)), _full_spec((1, G))],
        out_specs=[_row_spec(G), _row_spec(G), _row_spec(G)],
        out_shape=[out, out, out],
    )(ap, hp, degp, b, wpi_t, wmu_t, wls_t, bpi, bmu, bls)


def kernel(x, edge_index, W1, b1, W2, b2, Wpi, bpi, Wmu, bmu, Wls, bls):
    ei = edge_index.astype(jnp.int32)
    pad = EPAD - E
    src = jnp.concatenate([ei[0], jnp.zeros((pad,), jnp.int32)])
    dst = jnp.concatenate([ei[1], jnp.full((pad,), N, jnp.int32)])

    eidx = jnp.stack([src.reshape(NCHT, CHUNK),
                      dst.reshape(NCHT, CHUNK)], axis=1)

    deg_kernel, segsum_kernel = _sc_kernels()
    degt = deg_kernel(eidx).reshape(NW, NPAD)
    degp = _degsum(degt)[:N, None]
    h1p = _mm_scale(x, degp, W1.T)
    a1 = segsum_kernel(eidx, h1p)
    h2p = _layer_finish(a1, h1p, degp, b1.reshape(1, D), W2.T)
    a2 = segsum_kernel(eidx, h2p)
    pi, mu, ls = _heads(a2, h2p, degp, b2.reshape(1, D),
                        Wpi.T, Wmu.T, Wls.T,
                        bpi.reshape(1, G), bmu.reshape(1, G), bls.reshape(1, G))
    return (pi, mu, ls)


# speedup vs baseline: 1.2016x; 1.0270x over previous
import functools

import jax
import jax.numpy as jnp
from jax import lax
from jax.experimental import pallas as pl
from jax.experimental.pallas import tpu as pltpu
from jax.experimental.pallas import tpu_sc as plsc

N = 10000
D = 128
G = 8
E = 320000

NC, NS, L = 2, 16, 16
NW = NC * NS
CHUNK = 128
NCH = 79
EPW = NCH * CHUNK
EPAD = NW * EPW
NCHT = EPAD // CHUNK
NCH0 = 109
NCH1 = 2 * NCH - NCH0
NPAD = 10240
RPS = NPAD // NS

BM = 2000


def _deg_body(eidx_hbm, out_hbm, ib, acc_t):
    c = lax.axis_index("c")
    s = lax.axis_index("s")
    gw = c * NS + s

    pltpu.sync_copy(eidx_hbm.at[pl.ds(gw * NCH, NCH)], ib)

    one = jnp.ones((L,), jnp.float32)
    zero = jnp.zeros((L,), jnp.float32)

    def zero_acc(i, carry):
        acc_t[i, :] = zero
        return carry

    lax.fori_loop(0, NPAD // L, zero_acc, 0)

    def edge_body(g, carry):
        for k in range(CHUNK // L):
            idx = ib[g, 1, pl.ds(k * L, L)]
            row = lax.shift_right_logical(idx, 4)
            col = lax.bitwise_and(idx, 15)
            plsc.addupdate_scatter(acc_t, [row, col], one)
        return carry

    lax.fori_loop(0, NCH, edge_body, 0)

    pltpu.sync_copy(acc_t, out_hbm.at[gw])


def _segsum_body(eidx_hbm, h_hbm, out_hbm,
                 ib0, ib1, ib2, rows0, rows1, z_v, acc_sh,
                 sem_g0, sem_g1, sem_s0, sem_s1, sem_i0, sem_i1, sem_i2):
    c = lax.axis_index("c")
    s = lax.axis_index("s")
    gw = c * NS + s

    ib = (ib0, ib1, ib2)
    rows = (rows0, rows1)
    sem_g = (sem_g0, sem_g1)
    sem_s = (sem_s0, sem_s1)
    sem_i = (sem_i0, sem_i1, sem_i2)

    def gather_chunk(b, i3, sem_b):
        pltpu.async_copy(h_hbm.at[ib[i3].at[0]], rows[b], sem_b)

    def drain_gather(b, i3, sem_b):
        pltpu.make_async_copy(h_hbm.at[ib[i3].at[0]], rows[b], sem_b).wait()

    def scatter_chunk(b, i3, sem_b):
        pltpu.async_copy(rows[b], acc_sh.at[ib[i3].at[1]], sem_b, add=True)

    def drain_scatter(b, i3, sem_b):
        pltpu.make_async_copy(rows[b], acc_sh.at[ib[i3].at[1]], sem_b).wait()

    def fetch_idx(g, i3, sem_b):
        pltpu.async_copy(eidx_hbm.at[g], ib[i3], sem_b)

    def drain_idx(g, i3, sem_b):
        pltpu.make_async_copy(eidx_hbm.at[g], ib[i3], sem_b).wait()

    def run_pipeline(base, count):
        pltpu.sync_copy(eidx_hbm.at[base], ib[0])
        gather_chunk(0, 0, sem_g[0])
        fetch_idx(base + 1, 1, sem_i[1])

        zero = jnp.zeros((L,), jnp.float32)
        for i in range(16):
            for j in range(D // L):
                z_v[i, pl.ds(j * L, L)] = zero

        def zero_body(j, carry):
            pltpu.sync_copy(z_v, acc_sh.at[pl.ds(s * RPS + j * 16, 16)])
            return carry

        lax.fori_loop(0, RPS // 16, zero_body, 0)
        plsc.subcore_barrier()

        def half(t, b, i3):
            drain_gather(b, i3, sem_g[b])

            @pl.when(t > 0)
            def _():
                drain_scatter(1 - b, (i3 + 2) % 3, sem_s[1 - b])

            @pl.when(t + 1 < count)
            def _():
                drain_idx(base + t + 1, (i3 + 1) % 3, sem_i[(i3 + 1) % 3])
                gather_chunk(1 - b, (i3 + 1) % 3, sem_g[1 - b])

            scatter_chunk(b, i3, sem_s[b])

            @pl.when(t + 2 < count)
            def _():
                fetch_idx(base + t + 2, (i3 + 2) % 3, sem_i[(i3 + 2) % 3])

        def six(p, carry):
            for u in range(6):
                half(6 * p + u, u % 2, u % 3)
            return carry

        lax.fori_loop(0, count // 6, six, 0)
        half(count - 1, 0, 0)
        drain_scatter(0, 0, sem_s[0])
        plsc.subcore_barrier()

    @pl.when(c == 0)
    def _():
        run_pipeline(s * NCH0, NCH0)

    @pl.when(c == 1)
    def _():
        run_pipeline(NS * NCH0 + s * NCH1, NCH1)

    pltpu.sync_copy(acc_sh.at[pl.ds(s * RPS, RPS)],
                    out_hbm.at[c, pl.ds(s * RPS, RPS)])


@functools.cache
def _sc_kernels():
    mesh = plsc.VectorSubcoreMesh(core_axis_name="c", subcore_axis_name="s",
                                  num_cores=NC, num_subcores=NS)
    deg = pl.kernel(
        _deg_body,
        out_type=jax.ShapeDtypeStruct((NW, NPAD // L, L), jnp.float32),
        mesh=mesh,
        compiler_params=pltpu.CompilerParams(needs_layout_passes=False),
        scratch_types=[
            pltpu.VMEM((NCH, 2, CHUNK), jnp.int32),
            pltpu.VMEM((NPAD // L, L), jnp.float32),
        ],
    )
    seg = pl.kernel(
        _segsum_body,
        out_type=jax.ShapeDtypeStruct((NC, NPAD, D), jnp.float32),
        mesh=mesh,
        scratch_types=[
            pltpu.VMEM((2, CHUNK), jnp.int32),
            pltpu.VMEM((2, CHUNK), jnp.int32),
            pltpu.VMEM((2, CHUNK), jnp.int32),
            pltpu.VMEM((CHUNK, D), jnp.float32),
            pltpu.VMEM((CHUNK, D), jnp.float32),
            pltpu.VMEM((16, D), jnp.float32),
            pltpu.VMEM_SHARED((NPAD, D), jnp.float32),
            pltpu.SemaphoreType.DMA,
            pltpu.SemaphoreType.DMA,
            pltpu.SemaphoreType.DMA,
            pltpu.SemaphoreType.DMA,
            pltpu.SemaphoreType.DMA,
            pltpu.SemaphoreType.DMA,
            pltpu.SemaphoreType.DMA,
        ],
    )
    return deg, seg


def _degsum_body(dp_ref, o_ref):
    o_ref[...] = jnp.sum(dp_ref[...], axis=0)


def _degsum(degp):
    return pl.pallas_call(
        _degsum_body,
        out_shape=jax.ShapeDtypeStruct((NPAD,), jnp.float32),
    )(degp)


def _dis_from(dp_ref):
    deg = dp_ref[...] + 1.0
    return lax.rsqrt(deg)


def _mm_scale_body(x_ref, dp_ref, w_ref, o_ref):
    dis = _dis_from(dp_ref)
    o_ref[...] = jnp.dot(x_ref[...], w_ref[...],
                         preferred_element_type=jnp.float32) * dis


def _layer_body(ap_ref, hp_ref, dp_ref, b_ref, w_ref, o_ref):
    dis = _dis_from(dp_ref)
    t = jnp.maximum((ap_ref[0] + ap_ref[1] + hp_ref[...]) * dis + b_ref[...],
                    0.0)
    o_ref[...] = jnp.dot(t, w_ref[...],
                         preferred_element_type=jnp.float32) * dis


def _heads_body(ap_ref, hp_ref, dp_ref, b_ref,
                wpi_ref, wmu_ref, wls_ref, bpi_ref, bmu_ref, bls_ref,
                opi_ref, omu_ref, ols_ref):
    dis = _dis_from(dp_ref)
    h = jnp.maximum((ap_ref[0] + ap_ref[1] + hp_ref[...]) * dis + b_ref[...],
                    0.0)
    zpi = jnp.dot(h, wpi_ref[...], preferred_element_type=jnp.float32) + bpi_ref[...]
    m = jnp.max(zpi, axis=1, keepdims=True)
    e = jnp.exp(zpi - m)
    opi_ref[...] = e / jnp.sum(e, axis=1, keepdims=True)
    omu_ref[...] = jnp.dot(h, wmu_ref[...], preferred_element_type=jnp.float32) + bmu_ref[...]
    ols_ref[...] = jnp.dot(h, wls_ref[...], preferred_element_type=jnp.float32) + bls_ref[...]


def _row_spec(width):
    return pl.BlockSpec((BM, width), lambda j: (j, 0))


_dp_spec = pl.BlockSpec((BM, 1), lambda j: (j, 0))
_ap_spec = pl.BlockSpec((NC, BM, D), lambda j: (0, j, 0))


def _full_spec(shape):
    return pl.BlockSpec(shape, lambda j: tuple(0 for _ in shape))


def _mm_scale(x, degp, w_t):
    return pl.pallas_call(
        _mm_scale_body,
        grid=(N // BM,),
        in_specs=[_row_spec(D), _dp_spec, _full_spec((D, D))],
        out_specs=_row_spec(D),
        out_shape=jax.ShapeDtypeStruct((N, D), jnp.float32),
    )(x, degp, w_t)


def _layer_finish(ap, hp, degp, b, w_t):
    return pl.pallas_call(
        _layer_body,
        grid=(N // BM,),
        in_specs=[_ap_spec, _row_spec(D), _dp_spec,
                  _full_spec((1, D)), _full_spec((D, D))],
        out_specs=_row_spec(D),
        out_shape=jax.ShapeDtypeStruct((N, D), jnp.float32),
    )(ap, hp, degp, b, w_t)


def _heads(ap, hp, degp, b, wpi_t, wmu_t, wls_t, bpi, bmu, bls):
    out = jax.ShapeDtypeStruct((N, G), jnp.float32)
    return pl.pallas_call(
        _heads_body,
        grid=(N // BM,),
        in_specs=[_ap_spec, _row_spec(D), _dp_spec, _full_spec((1, D)),
                  _full_spec((D, G)), _full_spec((D, G)), _full_spec((D, G)),
                  _full_spec((1, G)), _full_spec((1, G)), _full_spec((1, G))],
        out_specs=[_row_spec(G), _row_spec(G), _row_spec(G)],
        out_shape=[out, out, out],
    )(ap, hp, degp, b, wpi_t, wmu_t, wls_t, bpi, bmu, bls)


def kernel(x, edge_index, W1, b1, W2, b2, Wpi, bpi, Wmu, bmu, Wls, bls):
    ei = edge_index.astype(jnp.int32)
    pad = EPAD - E
    src = jnp.concatenate([ei[0], jnp.zeros((pad,), jnp.int32)])
    dst = jnp.concatenate([ei[1], jnp.full((pad,), N, jnp.int32)])

    eidx = jnp.stack([src.reshape(NCHT, CHUNK),
                      dst.reshape(NCHT, CHUNK)], axis=1)

    deg_kernel, segsum_kernel = _sc_kernels()
    degt = deg_kernel(eidx).reshape(NW, NPAD)
    degp = _degsum(degt)[:N, None]
    h1p = _mm_scale(x, degp, W1.T)
    a1 = segsum_kernel(eidx, h1p)
    h2p = _layer_finish(a1, h1p, degp, b1.reshape(1, D), W2.T)
    a2 = segsum_kernel(eidx, h2p)
    pi, mu, ls = _heads(a2, h2p, degp, b2.reshape(1, D),
                        Wpi.T, Wmu.T, Wls.T,
                        bpi.reshape(1, G), bmu.reshape(1, G), bls.reshape(1, G))
    return (pi, mu, ls)
